# Initial kernel scaffold; baseline (speedup 1.0000x reference)
import jax
import jax.numpy as jnp
from jax.experimental import pallas as pl


def kernel(x, mapping):
    raise NotImplementedError("write your pallas kernel here")


# speedup vs baseline: 1.6287x; 1.6287x over previous
import functools

import jax
import jax.numpy as jnp
from jax import lax
from jax.experimental import pallas as pl
from jax.experimental.pallas import tpu as pltpu
from jax.experimental.pallas import tpu_sc as plsc

_ROWS_PER_BLOCK = 512
_LANES = 16


def _argmax_body(x_ref, idx_ref):
    xb = x_ref[...]
    m = jnp.max(xb, axis=1, keepdims=True)
    col = lax.broadcasted_iota(jnp.int32, xb.shape, 1)
    cand = jnp.where(xb == m, col, xb.shape[1])
    idx_ref[...] = jnp.min(cand, axis=1)


def _tc_argmax(x):
    n, c = x.shape
    r = _ROWS_PER_BLOCK
    return pl.pallas_call(
        _argmax_body,
        grid=(n // r,),
        in_specs=[pl.BlockSpec((r, c), lambda i: (i, 0))],
        out_specs=pl.BlockSpec((r,), lambda i: (i,)),
        out_shape=jax.ShapeDtypeStruct((n,), jnp.int32),
    )(x)


def _sc_lookup(table_flat, idx):
    n = idx.shape[0]
    info = plsc.get_sparse_core_info()
    nw = info.num_cores * info.num_subcores
    bpw = n // nw
    mesh = plsc.VectorSubcoreMesh(core_axis_name="c", subcore_axis_name="s")

    @functools.partial(
        pl.kernel,
        mesh=mesh,
        out_type=jax.ShapeDtypeStruct((2 * n,), jnp.float32),
        scratch_types=[
            pltpu.VMEM((table_flat.shape[0],), jnp.float32),
            pltpu.VMEM((bpw,), jnp.int32),
            pltpu.VMEM((2 * bpw,), jnp.float32),
        ],
        compiler_params=pltpu.CompilerParams(needs_layout_passes=False),
    )
    def _k(table_hbm, idx_hbm, out_hbm, tbl_v, idx_v, out_v):
        wid = lax.axis_index("s") * info.num_cores + lax.axis_index("c")
        base = wid * bpw
        pltpu.sync_copy(table_hbm, tbl_v)
        pltpu.sync_copy(idx_hbm.at[pl.ds(base, bpw)], idx_v)

        def body(i, carry):
            off = pl.multiple_of(i * _LANES, _LANES)
            iv = idx_v[pl.ds(off, _LANES)]
            first = plsc.load_gather(tbl_v, [iv * 2])
            second = plsc.load_gather(tbl_v, [iv * 2 + 1])
            pos = (lax.iota(jnp.int32, _LANES) + off) * 2
            plsc.store_scatter(out_v, [pos], first)
            plsc.store_scatter(out_v, [pos + 1], second)
            return carry

        lax.fori_loop(0, bpw // _LANES, body, 0)
        pltpu.sync_copy(out_v, out_hbm.at[pl.ds(2 * base, 2 * bpw)])

    return _k(table_flat, idx)


def kernel(x, mapping):
    idx = _tc_argmax(x)
    flat = _sc_lookup(mapping.reshape(-1), idx)
    return flat.reshape(x.shape[0], 2)


# speedup vs baseline: 2.3633x; 1.4510x over previous
import functools

import jax
import jax.numpy as jnp
from jax import lax
from jax.experimental import pallas as pl
from jax.experimental.pallas import tpu as pltpu
from jax.experimental.pallas import tpu_sc as plsc

_ROWS_PER_BLOCK = 2048
_LANES = 16


def _argmax_body(x_ref, idx_ref):
    xb = x_ref[...]
    m = jnp.max(xb, axis=1, keepdims=True)
    col = lax.broadcasted_iota(jnp.int32, xb.shape, 1)
    cand = jnp.where(xb == m, col, xb.shape[1])
    idx_ref[...] = jnp.min(cand, axis=1)


def _tc_argmax(x):
    n, c = x.shape
    r = _ROWS_PER_BLOCK
    return pl.pallas_call(
        _argmax_body,
        grid=(n // r,),
        in_specs=[pl.BlockSpec((r, c), lambda i: (i, 0))],
        out_specs=pl.BlockSpec((r,), lambda i: (i,)),
        out_shape=jax.ShapeDtypeStruct((n,), jnp.int32),
    )(x)


def _sc_lookup(table_flat, idx):
    n = idx.shape[0]
    info = plsc.get_sparse_core_info()
    nw = info.num_cores * info.num_subcores
    bpw = n // nw
    mesh = plsc.VectorSubcoreMesh(core_axis_name="c", subcore_axis_name="s")

    @functools.partial(
        pl.kernel,
        mesh=mesh,
        out_type=jax.ShapeDtypeStruct((2 * n,), jnp.float32),
        scratch_types=[
            pltpu.VMEM((table_flat.shape[0],), jnp.float32),
            pltpu.VMEM((bpw,), jnp.int32),
            pltpu.VMEM((2 * bpw,), jnp.float32),
        ],
        compiler_params=pltpu.CompilerParams(needs_layout_passes=False),
    )
    def _k(table_hbm, idx_hbm, out_hbm, tbl_v, idx_v, out_v):
        wid = lax.axis_index("s") * info.num_cores + lax.axis_index("c")
        base = wid * bpw
        pltpu.sync_copy(table_hbm, tbl_v)
        pltpu.sync_copy(idx_hbm.at[pl.ds(base, bpw)], idx_v)

        def body(i, carry):
            off = pl.multiple_of(i * _LANES, _LANES)
            iv = idx_v[pl.ds(off, _LANES)]
            first = plsc.load_gather(tbl_v, [iv * 2])
            second = plsc.load_gather(tbl_v, [iv * 2 + 1])
            pos = (lax.iota(jnp.int32, _LANES) + off) * 2
            plsc.store_scatter(out_v, [pos], first)
            plsc.store_scatter(out_v, [pos + 1], second)
            return carry

        lax.fori_loop(0, bpw // _LANES, body, 0)
        pltpu.sync_copy(out_v, out_hbm.at[pl.ds(2 * base, 2 * bpw)])

    return _k(table_flat, idx)


def kernel(x, mapping):
    idx = _tc_argmax(x)
    flat = _sc_lookup(mapping.reshape(-1), idx)
    return flat.reshape(x.shape[0], 2)


# speedup vs baseline: 2.4209x; 1.0244x over previous
import functools

import jax
import jax.numpy as jnp
from jax import lax
from jax.experimental import pallas as pl
from jax.experimental.pallas import tpu as pltpu
from jax.experimental.pallas import tpu_sc as plsc

_ROWS_PER_BLOCK = 8192
_LANES = 16


def _argmax_body(x_ref, idx_ref):
    xb = x_ref[...]
    m = jnp.max(xb, axis=1, keepdims=True)
    col = lax.broadcasted_iota(jnp.int32, xb.shape, 1)
    cand = jnp.where(xb == m, col, xb.shape[1])
    idx_ref[...] = jnp.min(cand, axis=1)


def _tc_argmax(x):
    n, c = x.shape
    r = _ROWS_PER_BLOCK
    return pl.pallas_call(
        _argmax_body,
        grid=(n // r,),
        in_specs=[pl.BlockSpec((r, c), lambda i: (i, 0))],
        out_specs=pl.BlockSpec((r,), lambda i: (i,)),
        out_shape=jax.ShapeDtypeStruct((n,), jnp.int32),
    )(x)


def _sc_lookup(table_flat, idx):
    n = idx.shape[0]
    info = plsc.get_sparse_core_info()
    nw = info.num_cores * info.num_subcores
    bpw = n // nw
    mesh = plsc.VectorSubcoreMesh(core_axis_name="c", subcore_axis_name="s")

    @functools.partial(
        pl.kernel,
        mesh=mesh,
        out_type=jax.ShapeDtypeStruct((2 * n,), jnp.float32),
        scratch_types=[
            pltpu.VMEM((table_flat.shape[0],), jnp.float32),
            pltpu.VMEM((bpw,), jnp.int32),
            pltpu.VMEM((2 * bpw,), jnp.float32),
        ],
        compiler_params=pltpu.CompilerParams(needs_layout_passes=False),
    )
    def _k(table_hbm, idx_hbm, out_hbm, tbl_v, idx_v, out_v):
        wid = lax.axis_index("s") * info.num_cores + lax.axis_index("c")
        base = wid * bpw
        pltpu.sync_copy(table_hbm, tbl_v)
        pltpu.sync_copy(idx_hbm.at[pl.ds(base, bpw)], idx_v)

        def body(i, carry):
            off = pl.multiple_of(i * _LANES, _LANES)
            iv = idx_v[pl.ds(off, _LANES)]
            first = plsc.load_gather(tbl_v, [iv * 2])
            second = plsc.load_gather(tbl_v, [iv * 2 + 1])
            pos = (lax.iota(jnp.int32, _LANES) + off) * 2
            plsc.store_scatter(out_v, [pos], first)
            plsc.store_scatter(out_v, [pos + 1], second)
            return carry

        lax.fori_loop(0, bpw // _LANES, body, 0)
        pltpu.sync_copy(out_v, out_hbm.at[pl.ds(2 * base, 2 * bpw)])

    return _k(table_flat, idx)


def kernel(x, mapping):
    idx = _tc_argmax(x)
    flat = _sc_lookup(mapping.reshape(-1), idx)
    return flat.reshape(x.shape[0], 2)
